# baseline (device time: 76632 ns/iter reference)
import jax
import jax.numpy as jnp
from jax import lax
from jax.experimental import pallas as pl
from jax.experimental.pallas import tpu as pltpu

SQ = 2048
D = 1024
HQ = 8
DH = 128
BLK = 64
SCALE = 0.08838834764831843
TILES = [(0, 256), (256, 256), (512, 256), (768, 256), (1024, 256),
         (1280, 256), (1536, 256), (1792, 128), (1920, 128)]
N_TILES = len(TILES)
NT = (((1,), (1,)), ((), ()))
NN = (((1,), (0,)), ((), ()))


def kernel(x, Wq, K_ext, V_ext, Wo):
    bf = jnp.bfloat16
    f32 = jnp.float32
    x2 = x.reshape(SQ, D).astype(bf)
    K2 = K_ext.reshape(SQ, D).astype(bf)
    V2 = V_ext.reshape(SQ, D).astype(bf)
    Wqb = Wq.astype(bf)
    Wob = Wo.astype(bf)

    def body(x_ref, wq_ref, k_ref, v_ref, wo_ref, out_ref,
             qbf, vaug, ctxbuf, send_sems, recv_sems, fwd_sems):
        my = lax.axis_index("i")

        def tile_ref(t):
            r0, rows = TILES[t]
            return ctxbuf.at[r0:r0 + rows, :]

        barrier = pltpu.get_barrier_semaphore()

        @pl.when((my == 1) | (my == 3))
        def _():
            pl.semaphore_signal(barrier, inc=1, device_id=(0,),
                                device_id_type=pl.DeviceIdType.MESH)
            pl.semaphore_wait(barrier, 1)

        @pl.when(my == 2)
        def _():
            for d in (1, 3):
                pl.semaphore_signal(barrier, inc=1, device_id=(d,),
                                    device_id_type=pl.DeviceIdType.MESH)

        @pl.when(my == 0)
        def _():
            pl.semaphore_wait(barrier, 2)

        @pl.when(my == 0)
        def _producer():
            qbf[...] = (jnp.dot(x_ref[...], wq_ref[...],
                                preferred_element_type=f32)
                        * SCALE).astype(bf)
            ones = jnp.ones((SQ, DH), bf)
            for h in range(HQ):
                vaug[:, h * 2 * DH:h * 2 * DH + DH] = \
                    v_ref[:, h * DH:(h + 1) * DH]
                vaug[:, h * 2 * DH + DH:(h + 1) * 2 * DH] = ones
            biases = {}
            for rows in {256, 128}:
                r = lax.broadcasted_iota(jnp.int32, (rows, rows), 0) // BLK
                c = lax.broadcasted_iota(jnp.int32, (rows, rows), 1) // BLK
                biases[rows] = jnp.where(r >= c, f32(0), f32(-1e9)).astype(bf)

            rdmas = []
            for t in range(N_TILES):
                r0, rows = TILES[t]
                for h in range(HQ):
                    hs = slice(h * DH, (h + 1) * DH)
                    vs = slice(h * 2 * DH, (h + 1) * 2 * DH)
                    qt = qbf[r0:r0 + rows, hs]
                    sd = lax.dot_general(
                        qt, k_ref[r0:r0 + rows, hs], NT,
                        preferred_element_type=f32)
                    wd = jnp.exp(sd.astype(bf) + biases[rows])
                    res = lax.dot_general(
                        wd, vaug[r0:r0 + rows, vs], NN,
                        preferred_element_type=f32)
                    if r0 > 0:
                        s1 = lax.dot_general(
                            qt, k_ref[0:r0, hs], NT,
                            preferred_element_type=f32)
                        w1 = jnp.exp(s1.astype(bf))
                        res = res + lax.dot_general(
                            w1, vaug[0:r0, vs], NN,
                            preferred_element_type=f32)
                    ctxbuf[r0:r0 + rows, hs] = \
                        (res[:, 0:DH] / res[:, DH:DH + 1]).astype(bf)
                for i, d in enumerate((1, 3)):
                    rd = pltpu.make_async_remote_copy(
                        src_ref=tile_ref(t), dst_ref=tile_ref(t),
                        send_sem=send_sems.at[t, i], recv_sem=recv_sems.at[t],
                        device_id=(d,), device_id_type=pl.DeviceIdType.MESH)
                    rd.start()
                    rdmas.append(rd)
            out_ref[...] = jnp.dot(ctxbuf[...], wo_ref[...],
                                   preferred_element_type=f32).astype(bf)
            for rd in rdmas:
                rd.wait_send()

        @pl.when(my != 0)
        def _consumer():
            fwd_conds = []
            for t in range(N_TILES):
                r0, rows = TILES[t]
                rd = pltpu.make_async_remote_copy(
                    src_ref=tile_ref(t), dst_ref=tile_ref(t),
                    send_sem=send_sems.at[t, 0], recv_sem=recv_sems.at[t],
                    device_id=(0,), device_id_type=pl.DeviceIdType.MESH)
                rd.wait_recv()
                cond = (my == 1) if t % 2 == 0 else (my == 3)
                fwd_conds.append(cond)

                @pl.when(cond)
                def _fwd(t=t):
                    f = pltpu.make_async_remote_copy(
                        src_ref=tile_ref(t), dst_ref=tile_ref(t),
                        send_sem=fwd_sems.at[t], recv_sem=recv_sems.at[t],
                        device_id=(2,), device_id_type=pl.DeviceIdType.MESH)
                    f.start()

                out_ref[r0:r0 + rows, :] = jnp.dot(
                    ctxbuf[r0:r0 + rows, :], wo_ref[...],
                    preferred_element_type=f32).astype(bf)

            for t in range(N_TILES):
                @pl.when(fwd_conds[t])
                def _fwd_wait(t=t):
                    f = pltpu.make_async_remote_copy(
                        src_ref=tile_ref(t), dst_ref=tile_ref(t),
                        send_sem=fwd_sems.at[t], recv_sem=recv_sems.at[t],
                        device_id=(2,), device_id_type=pl.DeviceIdType.MESH)
                    f.wait_send()

    out = pl.pallas_call(
        body,
        out_shape=jax.ShapeDtypeStruct((SQ, D), bf),
        in_specs=[pl.BlockSpec(memory_space=pltpu.VMEM)] * 5,
        out_specs=pl.BlockSpec(memory_space=pltpu.VMEM),
        scratch_shapes=[
            pltpu.VMEM((SQ, D), bf),
            pltpu.VMEM((SQ, 2 * D), bf),
            pltpu.VMEM((SQ, D), bf),
            pltpu.SemaphoreType.DMA((N_TILES, 2)),
            pltpu.SemaphoreType.DMA((N_TILES,)),
            pltpu.SemaphoreType.DMA((N_TILES,)),
        ],
        compiler_params=pltpu.CompilerParams(collective_id=0),
    )(x2, Wqb, K2, V2, Wob)

    return out.reshape(1, SQ, D)
